# Initial kernel scaffold; baseline (speedup 1.0000x reference)
import jax
import jax.numpy as jnp
from jax.experimental import pallas as pl


def kernel(inputs):
    raise NotImplementedError("write your pallas kernel here")


# speedup vs baseline: 8.3233x; 8.3233x over previous
import functools

import jax
import jax.numpy as jnp
from jax import lax
from jax.experimental import pallas as pl
from jax.experimental.pallas import tpu as pltpu
from jax.experimental.pallas import tpu_sc as plsc

B = 4
N = 8192
C = 16
NPOINT = 1024
NSAMPLE = 32

_NROW, _NCOL = 64, 128
_PROW, _PCOL = 8, 128
_MT = 128


def _fps_body(x_ref, y_ref, z_ref, nx_ref, ny_ref, nz_ref, fi_ref):
    iota_n = (lax.broadcasted_iota(jnp.int32, (_NROW, _NCOL), 0) * _NCOL
              + lax.broadcasted_iota(jnp.int32, (_NROW, _NCOL), 1))
    iota_p = (lax.broadcasted_iota(jnp.int32, (_PROW, _PCOL), 0) * _PCOL
              + lax.broadcasted_iota(jnp.int32, (_PROW, _PCOL), 1))

    def body(i, state):
        onehot_p = iota_p == i
        new_state = []
        for b in range(B):
            dists, far, idxs, nxa, nya, nza = state[b]
            x = x_ref[b]
            y = y_ref[b]
            z = z_ref[b]
            idxs = jnp.where(onehot_p, far, idxs)
            m = iota_n == far
            cx = jnp.sum(jnp.where(m, x, 0.0))
            cy = jnp.sum(jnp.where(m, y, 0.0))
            cz = jnp.sum(jnp.where(m, z, 0.0))
            nxa = jnp.where(onehot_p, cx, nxa)
            nya = jnp.where(onehot_p, cy, nya)
            nza = jnp.where(onehot_p, cz, nza)
            dx = x - cx
            dy = y - cy
            dz = z - cz
            d = dx * dx + dy * dy + dz * dz
            dists = jnp.minimum(dists, d)
            mx = jnp.max(dists)
            cand = jnp.where(dists == mx, iota_n, jnp.int32(N))
            far = jnp.min(cand)
            new_state.append((dists, far, idxs, nxa, nya, nza))
        return tuple(new_state)

    init = tuple(
        (jnp.full((_NROW, _NCOL), 1e10, jnp.float32),
         jnp.int32(0),
         jnp.zeros((_PROW, _PCOL), jnp.int32),
         jnp.zeros((_PROW, _PCOL), jnp.float32),
         jnp.zeros((_PROW, _PCOL), jnp.float32),
         jnp.zeros((_PROW, _PCOL), jnp.float32))
        for _ in range(B))
    final = lax.fori_loop(0, NPOINT, body, init)
    for b in range(B):
        _, _, idxs, nxa, nya, nza = final[b]
        nx_ref[b] = nxa
        ny_ref[b] = nya
        nz_ref[b] = nza
        fi_ref[b] = idxs


def _fps(xr, yr, zr, interpret=False):
    out = pl.pallas_call(
        _fps_body,
        out_shape=(
            jax.ShapeDtypeStruct((B, _PROW, _PCOL), jnp.float32),
            jax.ShapeDtypeStruct((B, _PROW, _PCOL), jnp.float32),
            jax.ShapeDtypeStruct((B, _PROW, _PCOL), jnp.float32),
            jax.ShapeDtypeStruct((B, _PROW, _PCOL), jnp.int32),
        ),
        interpret=interpret,
    )(xr, yr, zr)
    return out


def _knn_body(xyzt_ref, new_ref, idx_ref, gidx_ref, sq_ref):
    b = pl.program_id(0)
    xt = xyzt_ref[0]
    nw = new_ref[0]
    x = xt[0:1, :]
    y = xt[1:2, :]
    z = xt[2:3, :]
    n_xyz = x * x + y * y + z * z
    nwx = nw[:, 0:1]
    nwy = nw[:, 1:2]
    nwz = nw[:, 2:3]
    n_new = nwx * nwx + nwy * nwy + nwz * nwz
    def _r(v):
        return v.astype(jnp.bfloat16).astype(jnp.float32)
    dot = _r(nwx) * _r(x) + _r(nwy) * _r(y) + _r(nwz) * _r(z)
    sq_ref[:, :] = (n_new + n_xyz) - 2.0 * dot

    iota_c = lax.broadcasted_iota(jnp.int32, (_MT, N), 1)
    iota_s = lax.broadcasted_iota(jnp.int32, (_MT, NSAMPLE), 1)

    def jbody(j, acc):
        sq = sq_ref[:, :]
        mn = jnp.min(sq, axis=1, keepdims=True)
        cand = jnp.where(sq == mn, iota_c, jnp.int32(N))
        aj = jnp.min(cand, axis=1, keepdims=True)
        sq_ref[:, :] = jnp.where(iota_c == aj, jnp.float32(jnp.inf), sq)
        return jnp.where(iota_s == j, aj, acc)

    acc = lax.fori_loop(0, NSAMPLE, jbody,
                        jnp.zeros((_MT, NSAMPLE), jnp.int32))
    idx_ref[0] = acc
    gidx_ref[0] = acc + b * N


def _knn(xyzt, new_xyz, interpret=False):
    grid = (B, NPOINT // _MT)
    return pl.pallas_call(
        _knn_body,
        grid=grid,
        in_specs=[
            pl.BlockSpec((1, 3, N), lambda b, m: (b, 0, 0)),
            pl.BlockSpec((1, _MT, 3), lambda b, m: (b, m, 0)),
        ],
        out_specs=(
            pl.BlockSpec((1, _MT, NSAMPLE), lambda b, m: (b, m, 0)),
            pl.BlockSpec((1, _MT, NSAMPLE), lambda b, m: (b, m, 0)),
        ),
        out_shape=(
            jax.ShapeDtypeStruct((B, NPOINT, NSAMPLE), jnp.int32),
            jax.ShapeDtypeStruct((B, NPOINT, NSAMPLE), jnp.int32),
        ),
        scratch_shapes=[pltpu.VMEM((_MT, N), jnp.float32)],
        interpret=interpret,
    )(xyzt, new_xyz)


_NW = 32
_ROWS = B * NPOINT * NSAMPLE
_RPT = _ROWS // _NW
_CPT = (B * NPOINT) // _NW
_CHUNK = 128
_NCH = _RPT // _CHUNK


def _group_sc(table, gidx2, ctab):
    mesh = plsc.VectorSubcoreMesh(core_axis_name="c", subcore_axis_name="s")

    @functools.partial(
        pl.kernel,
        out_type=jax.ShapeDtypeStruct((_ROWS, C), jnp.float32),
        mesh=mesh,
        compiler_params=pltpu.CompilerParams(use_tc_tiling_on_sc=False),
        scratch_types=[
            pltpu.VMEM((_NCH, _CHUNK), jnp.int32),
            pltpu.VMEM((_RPT, C), jnp.float32),
            pltpu.VMEM((_CPT, C), jnp.float32),
            pltpu.SemaphoreType.DMA,
        ],
    )
    def k(table_hbm, gidx_hbm, ctab_hbm, out_hbm, idx_v, rows_v, cent_v, sem):
        wid = lax.axis_index("s") * 2 + lax.axis_index("c")
        pltpu.sync_copy(gidx_hbm.at[pl.ds(wid * _NCH, _NCH)], idx_v)
        pltpu.sync_copy(ctab_hbm.at[pl.ds(wid * _CPT, _CPT)], cent_v)
        for g in range(0, _NCH, 16):
            copies = [
                pltpu.async_copy(
                    table_hbm.at[idx_v.at[j]],
                    rows_v.at[pl.ds(j * _CHUNK, _CHUNK)],
                    sem,
                )
                for j in range(g, g + 16)
            ]
            for cp in copies:
                cp.wait()

        def sub_body(r, carry):
            cen = cent_v[lax.shift_right_logical(r, 5)]
            rows_v[r] = rows_v[r] - cen
            return carry

        lax.fori_loop(0, _RPT, sub_body, 0)
        pltpu.sync_copy(rows_v, out_hbm.at[pl.ds(wid * _RPT, _RPT)])

    return k(table, gidx2, ctab)


def kernel(inputs):
    xyz = inputs[:, :, 0:3]
    xr = inputs[:, :, 0].reshape(B, _NROW, _NCOL)
    yr = inputs[:, :, 1].reshape(B, _NROW, _NCOL)
    zr = inputs[:, :, 2].reshape(B, _NROW, _NCOL)

    nx, ny, nz, _ = _fps(xr, yr, zr)
    new_xyz = jnp.stack(
        [nx.reshape(B, NPOINT), ny.reshape(B, NPOINT), nz.reshape(B, NPOINT)],
        axis=-1)

    xyzt = xyz.transpose(0, 2, 1)
    idx, gidx = _knn(xyzt, new_xyz)

    table = inputs.reshape(B * N, C)
    gidx2 = gidx.reshape(_ROWS // _CHUNK, _CHUNK)
    ctab = jnp.concatenate(
        [new_xyz.reshape(B * NPOINT, 3),
         jnp.zeros((B * NPOINT, C - 3), jnp.float32)], axis=1)

    flat = _group_sc(table, gidx2, ctab)
    new_points = flat.reshape(B, NPOINT, NSAMPLE, C)
    grouped_xyz = new_points[..., 0:3]
    return new_xyz, new_points, idx, grouped_xyz
